# baseline (device time: 24903 ns/iter reference)
import jax
import jax.numpy as jnp
from jax import lax
from jax.experimental import pallas as pl
from jax.experimental.pallas import tpu as pltpu

N_X = 2
N_Y = 4
N_Z = 4
STRIPE = 64
HALF = 256


def kernel(x):
    m, n = x.shape

    def body(
        x_ref,
        out_ref,
        p1_buf,
        pz_buf,
        px_buf,
        zx_sem,
        s1_send,
        s1_recv,
        sz_send,
        sz_recv,
        sx_send,
        sx_recv,
    ):
        my_x = lax.axis_index("x")
        my_y = lax.axis_index("y")
        my_z = lax.axis_index("z")
        twin_x = 1 - my_x
        half = HALF * my_x
        base = half + STRIPE * my_z
        stripe_rows = pl.ds(base, STRIPE)

        barrier_sem = pltpu.get_barrier_semaphore()
        for d in range(1, N_Y):
            pl.semaphore_signal(
                barrier_sem, inc=1,
                device_id=(my_x, (my_y + d) % N_Y, my_z),
                device_id_type=pl.DeviceIdType.MESH,
            )
        for d in range(1, N_Z):
            pl.semaphore_signal(
                zx_sem, inc=1,
                device_id=(my_x, my_y, (my_z + d) % N_Z),
                device_id_type=pl.DeviceIdType.MESH,
            )
        pl.semaphore_signal(
            zx_sem, inc=1,
            device_id=(twin_x, my_y, my_z),
            device_id_type=pl.DeviceIdType.MESH,
        )
        pl.semaphore_wait(barrier_sem, N_Y - 1)

        p1 = []
        for d in range(1, N_Y):
            rdma = pltpu.make_async_remote_copy(
                src_ref=x_ref.at[stripe_rows, :],
                dst_ref=p1_buf.at[d - 1],
                send_sem=s1_send.at[d - 1],
                recv_sem=s1_recv.at[d - 1],
                device_id=(my_x, (my_y + d) % N_Y, my_z),
                device_id_type=pl.DeviceIdType.MESH,
            )
            rdma.start()
            p1.append(rdma)
        for rdma in p1:
            rdma.wait_recv()
        out_ref[stripe_rows, :] = (
            x_ref[stripe_rows, :] + p1_buf[0] + p1_buf[1] + p1_buf[2]
        )

        pl.semaphore_wait(zx_sem, (N_Z - 1) + 1)

        def x_forward(src, slot):
            rdma = pltpu.make_async_remote_copy(
                src_ref=src,
                dst_ref=px_buf.at[slot],
                send_sem=sx_send.at[slot],
                recv_sem=sx_recv.at[slot],
                device_id=(twin_x, my_y, my_z),
                device_id_type=pl.DeviceIdType.MESH,
            )
            rdma.start()
            return rdma

        px = [x_forward(out_ref.at[stripe_rows, :], 0)]

        pz = []
        for d in range(1, N_Z):
            rdma = pltpu.make_async_remote_copy(
                src_ref=out_ref.at[stripe_rows, :],
                dst_ref=pz_buf.at[d - 1],
                send_sem=sz_send.at[d - 1],
                recv_sem=sz_recv.at[d - 1],
                device_id=(my_x, my_y, (my_z + d) % N_Z),
                device_id_type=pl.DeviceIdType.MESH,
            )
            rdma.start()
            pz.append(rdma)

        for d, rdma in zip(range(1, N_Z), pz):
            rdma.wait_recv()
            src_z = (my_z - d) % N_Z
            out_ref[pl.ds(half + STRIPE * src_z, STRIPE), :] = pz_buf[d - 1]
            px.append(x_forward(pz_buf.at[d - 1], d))

        twin_half = HALF * twin_x
        for slot in range(N_Z):
            rdma = pltpu.make_async_remote_copy(
                src_ref=out_ref.at[stripe_rows, :],
                dst_ref=px_buf.at[slot],
                send_sem=sx_send.at[slot],
                recv_sem=sx_recv.at[slot],
                device_id=(twin_x, my_y, my_z),
                device_id_type=pl.DeviceIdType.MESH,
            )
            rdma.wait_recv()
            src_z = (my_z - slot) % N_Z
            out_ref[pl.ds(twin_half + STRIPE * src_z, STRIPE), :] = (
                px_buf[slot]
            )

        for rdma in p1 + pz + px:
            rdma.wait_send()

    return pl.pallas_call(
        body,
        out_shape=jax.ShapeDtypeStruct((m, n), x.dtype),
        in_specs=[pl.BlockSpec(memory_space=pltpu.VMEM)],
        out_specs=pl.BlockSpec(memory_space=pltpu.VMEM),
        scratch_shapes=[
            pltpu.VMEM((N_Y - 1, STRIPE, n), x.dtype),
            pltpu.VMEM((N_Z - 1, STRIPE, n), x.dtype),
            pltpu.VMEM((N_Z, STRIPE, n), x.dtype),
            pltpu.SemaphoreType.REGULAR,
            pltpu.SemaphoreType.DMA((N_Y - 1,)),
            pltpu.SemaphoreType.DMA((N_Y - 1,)),
            pltpu.SemaphoreType.DMA((N_Z - 1,)),
            pltpu.SemaphoreType.DMA((N_Z - 1,)),
            pltpu.SemaphoreType.DMA((N_Z,)),
            pltpu.SemaphoreType.DMA((N_Z,)),
        ],
        compiler_params=pltpu.CompilerParams(collective_id=0),
    )(x)


# device time: 21376 ns/iter; 1.1650x vs baseline; 1.1650x over previous
import jax
import jax.numpy as jnp
from jax import lax
from jax.experimental import pallas as pl
from jax.experimental.pallas import tpu as pltpu

N_X = 2
N_Y = 4
N_Z = 4
N_C = 2
CHUNK = 32
STRIPE = 64
HALF = 256


def kernel(x):
    m, n = x.shape

    def body(
        x_ref,
        out_ref,
        p1_buf,
        pz_buf,
        px_buf,
        zx_sem,
        s1_send,
        s1_recv,
        sz_send,
        sz_recv,
        sx_send,
        sx_recv,
    ):
        my_x = lax.axis_index("x")
        my_y = lax.axis_index("y")
        my_z = lax.axis_index("z")
        twin_x = 1 - my_x
        half = HALF * my_x
        base = half + STRIPE * my_z

        def my_chunk(c):
            return pl.ds(base + CHUNK * c, CHUNK)

        barrier_sem = pltpu.get_barrier_semaphore()
        for d in range(1, N_Y):
            pl.semaphore_signal(
                barrier_sem, inc=1,
                device_id=(my_x, (my_y + d) % N_Y, my_z),
                device_id_type=pl.DeviceIdType.MESH,
            )
        for d in range(1, N_Z):
            pl.semaphore_signal(
                zx_sem, inc=1,
                device_id=(my_x, my_y, (my_z + d) % N_Z),
                device_id_type=pl.DeviceIdType.MESH,
            )
        pl.semaphore_signal(
            zx_sem, inc=1,
            device_id=(twin_x, my_y, my_z),
            device_id_type=pl.DeviceIdType.MESH,
        )
        pl.semaphore_wait(barrier_sem, N_Y - 1)

        p1 = {}
        for c in range(N_C):
            for d in range(1, N_Y):
                rdma = pltpu.make_async_remote_copy(
                    src_ref=x_ref.at[my_chunk(c), :],
                    dst_ref=p1_buf.at[c, d - 1],
                    send_sem=s1_send.at[c, d - 1],
                    recv_sem=s1_recv.at[c, d - 1],
                    device_id=(my_x, (my_y + d) % N_Y, my_z),
                    device_id_type=pl.DeviceIdType.MESH,
                )
                rdma.start()
                p1[(c, d)] = rdma

        def x_forward(src, k, c):
            rdma = pltpu.make_async_remote_copy(
                src_ref=src,
                dst_ref=px_buf.at[k, c],
                send_sem=sx_send.at[k, c],
                recv_sem=sx_recv.at[k, c],
                device_id=(twin_x, my_y, my_z),
                device_id_type=pl.DeviceIdType.MESH,
            )
            rdma.start()
            return rdma

        pz = {}
        px = []
        for c in range(N_C):
            for d in range(1, N_Y):
                p1[(c, d)].wait_recv()
            rows = my_chunk(c)
            out_ref[rows, :] = (
                x_ref[rows, :]
                + p1_buf[c, 0]
                + p1_buf[c, 1]
                + p1_buf[c, 2]
            )
            if c == 0:
                pl.semaphore_wait(zx_sem, (N_Z - 1) + 1)
            px.append(x_forward(out_ref.at[rows, :], 0, c))
            for d in range(1, N_Z):
                rdma = pltpu.make_async_remote_copy(
                    src_ref=out_ref.at[rows, :],
                    dst_ref=pz_buf.at[c, d - 1],
                    send_sem=sz_send.at[c, d - 1],
                    recv_sem=sz_recv.at[c, d - 1],
                    device_id=(my_x, my_y, (my_z + d) % N_Z),
                    device_id_type=pl.DeviceIdType.MESH,
                )
                rdma.start()
                pz[(c, d)] = rdma

        for c in range(N_C):
            for d in range(1, N_Z):
                pz[(c, d)].wait_recv()
                src_z = (my_z - d) % N_Z
                rows = pl.ds(half + STRIPE * src_z + CHUNK * c, CHUNK)
                out_ref[rows, :] = pz_buf[c, d - 1]
                px.append(x_forward(pz_buf.at[c, d - 1], d, c))

        twin_half = HALF * twin_x
        order = [(0, 0), (0, 1)] + [
            (d, c) for c in range(N_C) for d in range(1, N_Z)
        ]
        for k, c in order:
            rdma = pltpu.make_async_remote_copy(
                src_ref=out_ref.at[my_chunk(0), :],
                dst_ref=px_buf.at[k, c],
                send_sem=sx_send.at[k, c],
                recv_sem=sx_recv.at[k, c],
                device_id=(twin_x, my_y, my_z),
                device_id_type=pl.DeviceIdType.MESH,
            )
            rdma.wait_recv()
            src_z = (my_z - k) % N_Z
            rows = pl.ds(twin_half + STRIPE * src_z + CHUNK * c, CHUNK)
            out_ref[rows, :] = px_buf[k, c]

        for rdma in list(p1.values()) + list(pz.values()) + px:
            rdma.wait_send()

    return pl.pallas_call(
        body,
        out_shape=jax.ShapeDtypeStruct((m, n), x.dtype),
        in_specs=[pl.BlockSpec(memory_space=pltpu.VMEM)],
        out_specs=pl.BlockSpec(memory_space=pltpu.VMEM),
        scratch_shapes=[
            pltpu.VMEM((N_C, N_Y - 1, CHUNK, n), x.dtype),
            pltpu.VMEM((N_C, N_Z - 1, CHUNK, n), x.dtype),
            pltpu.VMEM((N_Z, N_C, CHUNK, n), x.dtype),
            pltpu.SemaphoreType.REGULAR,
            pltpu.SemaphoreType.DMA((N_C, N_Y - 1)),
            pltpu.SemaphoreType.DMA((N_C, N_Y - 1)),
            pltpu.SemaphoreType.DMA((N_C, N_Z - 1)),
            pltpu.SemaphoreType.DMA((N_C, N_Z - 1)),
            pltpu.SemaphoreType.DMA((N_Z, N_C)),
            pltpu.SemaphoreType.DMA((N_Z, N_C)),
        ],
        compiler_params=pltpu.CompilerParams(collective_id=0),
    )(x)
